# baseline (speedup 1.0000x reference)
import jax
import jax.numpy as jnp
from jax.experimental import pallas as pl

_D_MODEL = 768
_D3 = _D_MODEL // 3


def _embed_add_kernel(tok_ref, x_ref, y_ref, t_ref, out_ref):
    tok = tok_ref[0, 0]
    x = x_ref[...]
    y = y_ref[...]
    t = t_ref[0]
    d = _D3
    out_ref[0, 0, :, :, 0:d] = tok[:, :, 0:d] + x[:, None, :]
    out_ref[0, 0, :, :, d:2 * d] = tok[:, :, d:2 * d] + y[None, :, :]
    out_ref[0, 0, :, :, 2 * d:3 * d] = tok[:, :, 2 * d:3 * d] + t[None, :, :]


def kernel(tokens, n_x, n_y, x_emb, y_emb, t_emb):
    B, tau, N, d = tokens.shape
    nx = x_emb.shape[0]
    ny = y_emb.shape[0]
    tok5 = tokens.reshape(B, tau, nx, ny, d)

    out5 = pl.pallas_call(
        _embed_add_kernel,
        grid=(B, tau),
        in_specs=[
            pl.BlockSpec((1, 1, nx, ny, d), lambda b, t: (b, t, 0, 0, 0)),
            pl.BlockSpec((nx, _D3), lambda b, t: (0, 0)),
            pl.BlockSpec((ny, _D3), lambda b, t: (0, 0)),
            pl.BlockSpec((1, 1, _D3), lambda b, t: (t, 0, 0)),
        ],
        out_specs=pl.BlockSpec((1, 1, nx, ny, d), lambda b, t: (b, t, 0, 0, 0)),
        out_shape=jax.ShapeDtypeStruct((B, tau, nx, ny, d), tokens.dtype),
    )(tok5, x_emb, y_emb, t_emb.reshape(tau, 1, _D3))

    return out5.reshape(B, tau, N, d)


# speedup vs baseline: 1.1706x; 1.1706x over previous
import jax
import jax.numpy as jnp
from jax.experimental import pallas as pl
from jax.experimental.pallas import tpu as pltpu

_D_MODEL = 768
_D3 = _D_MODEL // 3
_TAU_BLK = 4


def _embed_add_kernel(tok_ref, x_ref, y_ref, t_ref, out_ref):
    x = x_ref[...]
    y = y_ref[...]
    d = _D3
    for i in range(_TAU_BLK):
        tok = tok_ref[0, i]
        out_ref[0, i, :, :, 0:d] = tok[:, :, 0:d] + x[:, None, :]
        out_ref[0, i, :, :, d:2 * d] = tok[:, :, d:2 * d] + y[None, :, :]
        out_ref[0, i, :, :, 2 * d:3 * d] = tok[:, :, 2 * d:3 * d] + t_ref[i]


def kernel(tokens, n_x, n_y, x_emb, y_emb, t_emb):
    B, tau, N, d = tokens.shape
    nx = x_emb.shape[0]
    ny = y_emb.shape[0]
    tok5 = tokens.reshape(B, tau, nx, ny, d)

    out5 = pl.pallas_call(
        _embed_add_kernel,
        grid=(B, tau // _TAU_BLK),
        in_specs=[
            pl.BlockSpec((1, _TAU_BLK, nx, ny, d), lambda b, t: (b, t, 0, 0, 0)),
            pl.BlockSpec((nx, _D3), lambda b, t: (0, 0)),
            pl.BlockSpec((ny, _D3), lambda b, t: (0, 0)),
            pl.BlockSpec((_TAU_BLK, 1, _D3), lambda b, t: (t, 0, 0)),
        ],
        out_specs=pl.BlockSpec((1, _TAU_BLK, nx, ny, d), lambda b, t: (b, t, 0, 0, 0)),
        out_shape=jax.ShapeDtypeStruct((B, tau, nx, ny, d), tokens.dtype),
        compiler_params=pltpu.CompilerParams(
            dimension_semantics=("parallel", "parallel"),
        ),
    )(tok5, x_emb, y_emb, t_emb.reshape(tau, 1, _D3))

    return out5.reshape(B, tau, N, d)


# speedup vs baseline: 1.1806x; 1.0086x over previous
import jax
import jax.numpy as jnp
from jax.experimental import pallas as pl
from jax.experimental.pallas import tpu as pltpu

_D_MODEL = 768
_D3 = _D_MODEL // 3
_TAU_BLK = 8


def _embed_add_kernel(tok_ref, x_ref, y_ref, t_ref, out_ref):
    x = x_ref[...]
    y = y_ref[...]
    d = _D3
    for i in range(_TAU_BLK):
        tok = tok_ref[0, i]
        out_ref[0, i, :, :, 0:d] = tok[:, :, 0:d] + x[:, None, :]
        out_ref[0, i, :, :, d:2 * d] = tok[:, :, d:2 * d] + y[None, :, :]
        out_ref[0, i, :, :, 2 * d:3 * d] = tok[:, :, 2 * d:3 * d] + t_ref[i]


def kernel(tokens, n_x, n_y, x_emb, y_emb, t_emb):
    B, tau, N, d = tokens.shape
    nx = x_emb.shape[0]
    ny = y_emb.shape[0]
    tok5 = tokens.reshape(B, tau, nx, ny, d)

    out5 = pl.pallas_call(
        _embed_add_kernel,
        grid=(B, tau // _TAU_BLK),
        in_specs=[
            pl.BlockSpec((1, _TAU_BLK, nx, ny, d), lambda b, t: (b, t, 0, 0, 0)),
            pl.BlockSpec((nx, _D3), lambda b, t: (0, 0)),
            pl.BlockSpec((ny, _D3), lambda b, t: (0, 0)),
            pl.BlockSpec((_TAU_BLK, 1, _D3), lambda b, t: (t, 0, 0)),
        ],
        out_specs=pl.BlockSpec((1, _TAU_BLK, nx, ny, d), lambda b, t: (b, t, 0, 0, 0)),
        out_shape=jax.ShapeDtypeStruct((B, tau, nx, ny, d), tokens.dtype),
        compiler_params=pltpu.CompilerParams(
            dimension_semantics=("parallel", "parallel"),
            vmem_limit_bytes=128 * 1024 * 1024,
        ),
    )(tok5, x_emb, y_emb, t_emb.reshape(tau, 1, _D3))

    return out5.reshape(B, tau, N, d)
